# Initial kernel scaffold; baseline (speedup 1.0000x reference)
import jax
import jax.numpy as jnp
from jax.experimental import pallas as pl


def kernel(features, W_map, W_agg1, W_agg2, neigh1, neigh2):
    raise NotImplementedError("write your pallas kernel here")


# speedup vs baseline: 16.3977x; 16.3977x over previous
import functools

import jax
import jax.numpy as jnp
from jax import lax
from jax.experimental import pallas as pl
from jax.experimental.pallas import tpu as pltpu
from jax.experimental.pallas import tpu_sc as plsc

D = 32
NW = 32
N_PAD = 100352
PER_W = N_PAD // NW


def _map_body(x_ref, w_ref, o_ref):
    o_ref[...] = jnp.dot(x_ref[...], w_ref[...],
                         preferred_element_type=jnp.float32)


def _feature_map(features, w_t):
    n, f = features.shape
    b = 2000
    return pl.pallas_call(
        _map_body,
        grid=(n // b,),
        in_specs=[pl.BlockSpec((b, f), lambda i: (i, 0)),
                  pl.BlockSpec((f, D), lambda i: (0, 0))],
        out_specs=pl.BlockSpec((b, D), lambda i: (i, 0)),
        out_shape=jax.ShapeDtypeStruct((n, D), jnp.float32),
    )(features, w_t)


def _layer_body(e_ref, s_ref, w_ref, o_ref, *, inv):
    agg = (e_ref[...] + s_ref[...]) * inv
    h = jnp.maximum(jnp.dot(agg, w_ref[...],
                            preferred_element_type=jnp.float32), 0.0)
    nrm = jnp.sqrt(jnp.sum(h * h, axis=-1, keepdims=True))
    o_ref[...] = h / jnp.maximum(nrm, 1e-12)


def _dense_layer(emb, sums, w_t, inv):
    n = emb.shape[0]
    b = 2000
    return pl.pallas_call(
        functools.partial(_layer_body, inv=inv),
        grid=(n // b,),
        in_specs=[pl.BlockSpec((b, D), lambda i: (i, 0)),
                  pl.BlockSpec((b, D), lambda i: (i, 0)),
                  pl.BlockSpec((D, D), lambda i: (0, 0))],
        out_specs=pl.BlockSpec((b, D), lambda i: (i, 0)),
        out_shape=jax.ShapeDtypeStruct((n, D), jnp.float32),
    )(emb, sums, w_t)


def _gather_sum(emb, idx_flat, s_fan, chunk):
    cs = chunk * s_fan
    n_chunks = PER_W // chunk
    mesh = plsc.VectorSubcoreMesh(core_axis_name="c", subcore_axis_name="s",
                                  num_cores=2, num_subcores=16)

    def body(emb_hbm, idx_hbm, out_hbm, idx_v, rows_v, acc_v, sem):
        wid = lax.axis_index("s") * 2 + lax.axis_index("c")

        def chunk_body(ci, carry):
            node_off = wid * PER_W + ci * chunk
            pltpu.sync_copy(idx_hbm.at[pl.ds(node_off * s_fan, cs)], idx_v)
            pltpu.async_copy(emb_hbm.at[idx_v], rows_v, sem).wait()

            def accum(b, c2):
                k = b * s_fan
                lo = rows_v[k, pl.ds(0, 16)]
                hi = rows_v[k, pl.ds(16, 16)]
                for s in range(1, s_fan):
                    lo = lo + rows_v[k + s, pl.ds(0, 16)]
                    hi = hi + rows_v[k + s, pl.ds(16, 16)]
                acc_v[b, pl.ds(0, 16)] = lo
                acc_v[b, pl.ds(16, 16)] = hi
                return c2

            lax.fori_loop(0, chunk, accum, 0)
            pltpu.sync_copy(acc_v, out_hbm.at[pl.ds(node_off, chunk)])
            return carry

        lax.fori_loop(0, n_chunks, chunk_body, 0)

    run = pl.kernel(
        body,
        out_type=jax.ShapeDtypeStruct((N_PAD, D), jnp.float32),
        mesh=mesh,
        compiler_params=pltpu.CompilerParams(use_tc_tiling_on_sc=False),
        scratch_types=[
            pltpu.VMEM((cs,), jnp.int32),
            pltpu.VMEM((cs, D), jnp.float32),
            pltpu.VMEM((chunk, D), jnp.float32),
            pltpu.SemaphoreType.DMA,
        ],
    )
    return run(emb, idx_flat)


def kernel(features, W_map, W_agg1, W_agg2, neigh1, neigh2):
    n = features.shape[0]
    s1 = neigh1.shape[1]
    s2 = neigh2.shape[1]
    pad = N_PAD - n

    emb = _feature_map(features, W_map.T)
    n1 = jnp.pad(neigh1.astype(jnp.int32), ((0, pad), (0, 0))).reshape(-1)
    n2 = jnp.pad(neigh2.astype(jnp.int32), ((0, pad), (0, 0))).reshape(-1)

    sum1 = _gather_sum(emb, n1, s1, 112)[:n]
    emb1 = _dense_layer(emb, sum1, W_agg1.T, 1.0 / (s1 + 1.0))
    sum2 = _gather_sum(emb1, n2, s2, 112)[:n]
    emb2 = _dense_layer(emb1, sum2, W_agg2.T, 1.0 / (s2 + 1.0))
    return emb2


# speedup vs baseline: 18.6367x; 1.1365x over previous
import functools

import jax
import jax.numpy as jnp
from jax import lax
from jax.experimental import pallas as pl
from jax.experimental.pallas import tpu as pltpu
from jax.experimental.pallas import tpu_sc as plsc

D = 32
NW = 32
N_PAD = 100352
PER_W = N_PAD // NW


def _map_body(x_ref, w_ref, o_ref):
    o_ref[...] = jnp.dot(x_ref[...], w_ref[...],
                         preferred_element_type=jnp.float32)


def _feature_map(features, w_t):
    n, f = features.shape
    b = 2000
    return pl.pallas_call(
        _map_body,
        grid=(n // b,),
        in_specs=[pl.BlockSpec((b, f), lambda i: (i, 0)),
                  pl.BlockSpec((f, D), lambda i: (0, 0))],
        out_specs=pl.BlockSpec((b, D), lambda i: (i, 0)),
        out_shape=jax.ShapeDtypeStruct((n, D), jnp.float32),
    )(features, w_t)


def _layer_body(e_ref, s_ref, w_ref, o_ref, *, inv):
    agg = (e_ref[...] + s_ref[...]) * inv
    h = jnp.maximum(jnp.dot(agg, w_ref[...],
                            preferred_element_type=jnp.float32), 0.0)
    nrm = jnp.sqrt(jnp.sum(h * h, axis=-1, keepdims=True))
    o_ref[...] = h / jnp.maximum(nrm, 1e-12)


def _dense_layer(emb, sums, w_t, inv):
    n = emb.shape[0]
    b = 2000
    return pl.pallas_call(
        functools.partial(_layer_body, inv=inv),
        grid=(n // b,),
        in_specs=[pl.BlockSpec((b, D), lambda i: (i, 0)),
                  pl.BlockSpec((b, D), lambda i: (i, 0)),
                  pl.BlockSpec((D, D), lambda i: (0, 0))],
        out_specs=pl.BlockSpec((b, D), lambda i: (i, 0)),
        out_shape=jax.ShapeDtypeStruct((n, D), jnp.float32),
    )(emb, sums, w_t)


def _gather_sum(emb, idx_flat, s_fan, chunk):
    cs = chunk * s_fan
    n_chunks = PER_W // chunk
    assert n_chunks % 2 == 0
    mesh = plsc.VectorSubcoreMesh(core_axis_name="c", subcore_axis_name="s",
                                  num_cores=2, num_subcores=16)

    def body(emb_hbm, idx_hbm, out_hbm,
             idx_a, idx_b, rows_a, rows_b, acc_a, acc_b,
             sem_a, sem_b, osem_a, osem_b):
        wid = lax.axis_index("s") * 2 + lax.axis_index("c")
        base = wid * PER_W

        def start(idx_v, rows_v, sem, ci):
            pltpu.sync_copy(idx_hbm.at[pl.ds((base + ci * chunk) * s_fan, cs)],
                            idx_v)
            pltpu.async_copy(emb_hbm.at[idx_v], rows_v, sem)

        def accum(rows_v, acc_v):
            def node(b, c2):
                k = b * s_fan
                lo = rows_v[k, pl.ds(0, 16)]
                hi = rows_v[k, pl.ds(16, 16)]
                for s in range(1, s_fan):
                    lo = lo + rows_v[k + s, pl.ds(0, 16)]
                    hi = hi + rows_v[k + s, pl.ds(16, 16)]
                acc_v[b, pl.ds(0, 16)] = lo
                acc_v[b, pl.ds(16, 16)] = hi
                return c2
            lax.fori_loop(0, chunk, node, 0)

        def finish(idx_v, rows_v, sem, acc_v, osem, ci, p):
            pltpu.make_async_copy(emb_hbm.at[idx_v], rows_v, sem).wait()

            @pl.when(p > 0)
            def _():
                pltpu.make_async_copy(
                    acc_v, out_hbm.at[pl.ds(base + (ci - 2) * chunk, chunk)],
                    osem).wait()

            accum(rows_v, acc_v)
            pltpu.async_copy(
                acc_v, out_hbm.at[pl.ds(base + ci * chunk, chunk)], osem)

        start(idx_a, rows_a, sem_a, 0)

        def pair(p, carry):
            c0 = 2 * p
            start(idx_b, rows_b, sem_b, c0 + 1)
            finish(idx_a, rows_a, sem_a, acc_a, osem_a, c0, p)

            @pl.when(c0 + 2 < n_chunks)
            def _():
                start(idx_a, rows_a, sem_a, c0 + 2)

            finish(idx_b, rows_b, sem_b, acc_b, osem_b, c0 + 1, p)
            return carry

        lax.fori_loop(0, n_chunks // 2, pair, 0)
        pltpu.make_async_copy(
            acc_a, out_hbm.at[pl.ds(base + (n_chunks - 2) * chunk, chunk)],
            osem_a).wait()
        pltpu.make_async_copy(
            acc_b, out_hbm.at[pl.ds(base + (n_chunks - 1) * chunk, chunk)],
            osem_b).wait()

    run = pl.kernel(
        body,
        out_type=jax.ShapeDtypeStruct((N_PAD, D), jnp.float32),
        mesh=mesh,
        compiler_params=pltpu.CompilerParams(use_tc_tiling_on_sc=False),
        scratch_types=[
            pltpu.VMEM((cs,), jnp.int32),
            pltpu.VMEM((cs,), jnp.int32),
            pltpu.VMEM((cs, D), jnp.float32),
            pltpu.VMEM((cs, D), jnp.float32),
            pltpu.VMEM((chunk, D), jnp.float32),
            pltpu.VMEM((chunk, D), jnp.float32),
            pltpu.SemaphoreType.DMA,
            pltpu.SemaphoreType.DMA,
            pltpu.SemaphoreType.DMA,
            pltpu.SemaphoreType.DMA,
        ],
    )
    return run(emb, idx_flat)


def kernel(features, W_map, W_agg1, W_agg2, neigh1, neigh2):
    n = features.shape[0]
    s1 = neigh1.shape[1]
    s2 = neigh2.shape[1]
    pad = N_PAD - n

    emb = _feature_map(features, W_map.T)
    n1 = jnp.pad(neigh1.astype(jnp.int32), ((0, pad), (0, 0))).reshape(-1)
    n2 = jnp.pad(neigh2.astype(jnp.int32), ((0, pad), (0, 0))).reshape(-1)

    sum1 = _gather_sum(emb, n1, s1, 56)[:n]
    emb1 = _dense_layer(emb, sum1, W_agg1.T, 1.0 / (s1 + 1.0))
    sum2 = _gather_sum(emb1, n2, s2, 112)[:n]
    emb2 = _dense_layer(emb1, sum2, W_agg2.T, 1.0 / (s2 + 1.0))
    return emb2


# speedup vs baseline: 35.4883x; 1.9042x over previous
import functools

import jax
import jax.numpy as jnp
from jax import lax
from jax.experimental import pallas as pl
from jax.experimental.pallas import tpu as pltpu
from jax.experimental.pallas import tpu_sc as plsc

D = 32
NW = 32


def _map_body(x_ref, w_ref, o_ref):
    o_ref[...] = jnp.dot(x_ref[...], w_ref[...],
                         preferred_element_type=jnp.float32)


def _feature_map(features4, w_big):
    n4, f4 = features4.shape
    b4 = 1000
    return pl.pallas_call(
        _map_body,
        grid=(n4 // b4,),
        in_specs=[pl.BlockSpec((b4, f4), lambda i: (i, 0)),
                  pl.BlockSpec((f4, 4 * D), lambda i: (0, 0))],
        out_specs=pl.BlockSpec((b4, 4 * D), lambda i: (i, 0)),
        out_shape=jax.ShapeDtypeStruct((n4, 4 * D), jnp.float32),
    )(features4, w_big)


def _layer_body(e_ref, s_ref, w_ref, m_ref, mt_ref, o_ref, *, inv):
    agg = (e_ref[...] + s_ref[...]) * inv
    h = jnp.maximum(jnp.dot(agg, w_ref[...],
                            preferred_element_type=jnp.float32), 0.0)
    seg = jnp.dot(h * h, m_ref[...], preferred_element_type=jnp.float32)
    inv_n = 1.0 / jnp.maximum(jnp.sqrt(seg), 1e-12)
    scale = jnp.dot(inv_n, mt_ref[...], preferred_element_type=jnp.float32)
    o_ref[...] = h * scale


def _dense_layer(emb_p, sums_p, w_bd, m_seg, inv):
    n4 = emb_p.shape[0]
    b4 = 1000
    return pl.pallas_call(
        functools.partial(_layer_body, inv=inv),
        grid=(n4 // b4,),
        in_specs=[pl.BlockSpec((b4, 4 * D), lambda i: (i, 0)),
                  pl.BlockSpec((b4, 4 * D), lambda i: (i, 0)),
                  pl.BlockSpec((4 * D, 4 * D), lambda i: (0, 0)),
                  pl.BlockSpec((4 * D, 4), lambda i: (0, 0)),
                  pl.BlockSpec((4, 4 * D), lambda i: (0, 0))],
        out_specs=pl.BlockSpec((b4, 4 * D), lambda i: (i, 0)),
        out_shape=jax.ShapeDtypeStruct((n4, 4 * D), jnp.float32),
    )(emb_p, sums_p, w_bd, m_seg, m_seg.T)


def _gather_sum(emb2d, idx_flat, s_fan, chunk):
    n = emb2d.shape[0]
    cs = chunk * s_fan
    n_chunks = (n + chunk - 1) // chunk
    last_off = n - chunk
    mesh = plsc.VectorSubcoreMesh(core_axis_name="c", subcore_axis_name="s",
                                  num_cores=2, num_subcores=16)

    def body(emb_hbm, idx_hbm, out_hbm,
             idx_a, idx_b, rows_a, rows_b, acc_a, acc_b,
             sem_a, sem_b, osem_a, osem_b):
        wid = lax.axis_index("s") * 2 + lax.axis_index("c")
        cnt = (n_chunks - 1 - wid) // NW + 1
        pairs = cnt // 2

        def node_off(j):
            return jnp.minimum((wid + NW * j) * chunk, last_off)

        def start(idx_v, rows_v, sem, j):
            off = node_off(j)
            pltpu.sync_copy(idx_hbm.at[pl.ds(off * s_fan, cs)], idx_v)
            pltpu.async_copy(emb_hbm.at[idx_v], rows_v, sem)

        def accum(rows_v, acc_v):
            def node(b, c2):
                k = b * s_fan
                lo = rows_v[k, pl.ds(0, 16)]
                hi = rows_v[k, pl.ds(16, 16)]
                for s in range(1, s_fan):
                    lo = lo + rows_v[k + s, pl.ds(0, 16)]
                    hi = hi + rows_v[k + s, pl.ds(16, 16)]
                r = b // 4
                c = (b % 4) * D
                acc_v[r, pl.ds(c, 16)] = lo
                acc_v[r, pl.ds(c + 16, 16)] = hi
                return c2
            lax.fori_loop(0, chunk, node, 0)

        def finish(idx_v, rows_v, sem, acc_v, osem, j, wait_prev):
            pltpu.make_async_copy(emb_hbm.at[idx_v], rows_v, sem).wait()
            dst = out_hbm.at[pl.ds(node_off(j) * D // 128, chunk * D // 128)]

            @pl.when(wait_prev)
            def _():
                pltpu.make_async_copy(acc_v, dst, osem).wait()

            accum(rows_v, acc_v)
            pltpu.async_copy(acc_v, dst, osem)

        start(idx_a, rows_a, sem_a, 0)

        def pair(p, carry):
            start(idx_b, rows_b, sem_b, 2 * p + 1)
            finish(idx_a, rows_a, sem_a, acc_a, osem_a, 2 * p, p > 0)

            @pl.when(2 * p + 2 < cnt)
            def _():
                start(idx_a, rows_a, sem_a, 2 * p + 2)

            finish(idx_b, rows_b, sem_b, acc_b, osem_b, 2 * p + 1, p > 0)
            return carry

        lax.fori_loop(0, pairs, pair, 0)

        @pl.when(cnt % 2 == 1)
        def _():
            finish(idx_a, rows_a, sem_a, acc_a, osem_a, cnt - 1, pairs > 0)

        pltpu.make_async_copy(
            acc_a, out_hbm.at[pl.ds(0, chunk * D // 128)], osem_a).wait()
        pltpu.make_async_copy(
            acc_b, out_hbm.at[pl.ds(0, chunk * D // 128)], osem_b).wait()

    run = pl.kernel(
        body,
        out_type=jax.ShapeDtypeStruct((n * D // 128, 128), jnp.float32),
        mesh=mesh,
        compiler_params=pltpu.CompilerParams(use_tc_tiling_on_sc=False),
        scratch_types=[
            pltpu.VMEM((cs,), jnp.int32),
            pltpu.VMEM((cs,), jnp.int32),
            pltpu.VMEM((cs, D), jnp.float32),
            pltpu.VMEM((cs, D), jnp.float32),
            pltpu.VMEM((chunk * D // 128, 128), jnp.float32),
            pltpu.VMEM((chunk * D // 128, 128), jnp.float32),
            pltpu.SemaphoreType.DMA,
            pltpu.SemaphoreType.DMA,
            pltpu.SemaphoreType.DMA,
            pltpu.SemaphoreType.DMA,
        ],
    )
    return run(emb2d, idx_flat)


def kernel(features, W_map, W_agg1, W_agg2, neigh1, neigh2):
    n = features.shape[0]
    s1 = neigh1.shape[1]
    s2 = neigh2.shape[1]
    eye4 = jnp.eye(4, dtype=jnp.float32)

    features4 = features.reshape(n // 4, 4 * features.shape[1])
    w_big = jnp.kron(eye4, W_map.T.astype(jnp.float32))
    emb_p = _feature_map(features4, w_big)

    m_seg = jnp.kron(eye4, jnp.ones((D, 1), jnp.float32))
    n1 = neigh1.astype(jnp.int32).reshape(-1)
    n2 = neigh2.astype(jnp.int32).reshape(-1)

    sum1_p = _gather_sum(emb_p.reshape(n, D), n1, s1, 64)
    emb1_p = _dense_layer(emb_p, sum1_p, jnp.kron(eye4, W_agg1.T), m_seg,
                          1.0 / (s1 + 1.0))
    sum2_p = _gather_sum(emb1_p.reshape(n, D), n2, s2, 160)
    out_p = _dense_layer(emb1_p, sum2_p, jnp.kron(eye4, W_agg2.T), m_seg,
                         1.0 / (s2 + 1.0))
    return out_p.reshape(n, D)
